# Initial kernel scaffold; baseline (speedup 1.0000x reference)
import jax
import jax.numpy as jnp
from jax.experimental import pallas as pl


def kernel(x, edge_index, W1_self, W1_neigh, b1, W2_self, W2_neigh, b2, fc1_w, fc1_b, fc2_w, fc2_b, fc3_w, fc3_b):
    raise NotImplementedError("write your pallas kernel here")


# speedup vs baseline: 6.3314x; 6.3314x over previous
import functools

import jax
import jax.numpy as jnp
import numpy as np
from jax import lax
from jax.experimental import pallas as pl
from jax.experimental.pallas import tpu as pltpu
from jax.experimental.pallas import tpu_sc as plsc

N_NODES = 10000
N_EDGES = 320000
D = 128
DW = 144

NC = 2
NS = 16
NW = NC * NS

K = 128
EPW = 10112
EPAD = EPW * NW
NCHUNK = EPW // K
NPAD = 10240
RPT = NPAD // NS

_mesh = plsc.VectorSubcoreMesh(core_axis_name="c", subcore_axis_name="s")


def _make_seg_sum(width):
    @functools.partial(
        pl.kernel,
        mesh=_mesh,
        out_type=jax.ShapeDtypeStruct((NC, NPAD, width), jnp.float32),
        scratch_types=[
            pltpu.VMEM_SHARED((NPAD, width), jnp.float32),
            pltpu.VMEM((K,), jnp.int32),
            pltpu.VMEM((K,), jnp.int32),
            pltpu.VMEM((K, width), jnp.float32),
            pltpu.VMEM((16, width), jnp.float32),
            pltpu.SemaphoreType.DMA,
        ],
        compiler_params=pltpu.CompilerParams(use_tc_tiling_on_sc=False),
    )
    def seg_sum(src_hbm, dst_hbm, tab_hbm, agg_out,
                acc, idx_s, idx_d, rows, zblk, sem):
        c = lax.axis_index("c")
        s = lax.axis_index("s")
        wid = c * NS + s

        zero16 = jnp.zeros((16,), jnp.float32)

        def init_zrow(r, _):
            for j in range(width // 16):
                zblk[r, pl.ds(j * 16, 16)] = zero16
            return 0

        lax.fori_loop(0, 16, init_zrow, 0)

        base_r = s * RPT

        def zero_chunk(i, _):
            pltpu.sync_copy(zblk, acc.at[pl.ds(base_r + i * 16, 16)])
            return 0

        lax.fori_loop(0, RPT // 16, zero_chunk, 0)
        plsc.subcore_barrier()

        base_e = wid * EPW

        def chunk(j, _):
            off = base_e + j * K
            pltpu.sync_copy(src_hbm.at[pl.ds(off, K)], idx_s)
            pltpu.sync_copy(dst_hbm.at[pl.ds(off, K)], idx_d)
            pltpu.async_copy(tab_hbm.at[idx_s], rows, sem).wait()
            pltpu.sync_copy(rows, acc.at[idx_d], add=True)
            return 0

        lax.fori_loop(0, NCHUNK, chunk, 0)
        plsc.subcore_barrier()

        pltpu.sync_copy(acc.at[pl.ds(base_r, RPT)],
                        agg_out.at[c, pl.ds(base_r, RPT)])

    return seg_sum


_seg_sum_aug = _make_seg_sum(DW)
_seg_sum = _make_seg_sum(D)


_RB = 1000
_NB = N_NODES // _RB


def _sage_body(x_ref, a_ref, ws_ref, wn_ref, b_ref, o_ref):
    deg = jnp.maximum(a_ref[0, :, D:D + 1] + a_ref[1, :, D:D + 1], 1.0)
    mean = (a_ref[0, :, :D] + a_ref[1, :, :D]) / deg
    h = (jnp.dot(x_ref[...], ws_ref[...], preferred_element_type=jnp.float32)
         + jnp.dot(mean, wn_ref[...], preferred_element_type=jnp.float32)
         + b_ref[...])
    o_ref[...] = jnp.maximum(h, 0.0)


def _sage_dense(x, aggp, Ws, Wn, b):
    return pl.pallas_call(
        _sage_body,
        grid=(_NB,),
        in_specs=[
            pl.BlockSpec((_RB, D), lambda i: (i, 0)),
            pl.BlockSpec((NC, _RB, DW), lambda i: (0, i, 0)),
            pl.BlockSpec((D, D), lambda i: (0, 0)),
            pl.BlockSpec((D, D), lambda i: (0, 0)),
            pl.BlockSpec((1, D), lambda i: (0, 0)),
        ],
        out_specs=pl.BlockSpec((_RB, D), lambda i: (i, 0)),
        out_shape=jax.ShapeDtypeStruct((N_NODES, D), jnp.float32),
    )(x, aggp, Ws, Wn, b)


def _head_body(x_ref, a_ref, d_ref, ws_ref, wn_ref, b_ref,
               f1_ref, f1b_ref, f2_ref, f2b_ref, f3_ref, f3b_ref,
               o_ref, acc_ref):
    i = pl.program_id(0)
    deg = jnp.maximum(d_ref[0, :, D:D + 1] + d_ref[1, :, D:D + 1], 1.0)
    mean = (a_ref[0] + a_ref[1]) / deg
    h = (jnp.dot(x_ref[...], ws_ref[...], preferred_element_type=jnp.float32)
         + jnp.dot(mean, wn_ref[...], preferred_element_type=jnp.float32)
         + b_ref[...])
    h = jnp.maximum(h, 0.0)

    @pl.when(i == 0)
    def _():
        acc_ref[...] = jnp.zeros_like(acc_ref)

    acc_ref[...] += jnp.sum(h, axis=0, keepdims=True)

    @pl.when(i == _NB - 1)
    def _():
        pooled = acc_ref[...] * (1.0 / N_NODES)
        z = jnp.clip(
            jnp.dot(pooled, f1_ref[...], preferred_element_type=jnp.float32)
            + f1b_ref[...], 0.0, 6.0)
        z = jnp.clip(
            jnp.dot(z, f2_ref[...], preferred_element_type=jnp.float32)
            + f2b_ref[...], 0.0, 6.0)
        o_ref[...] = (jnp.dot(z, f3_ref[...], preferred_element_type=jnp.float32)
                      + f3b_ref[...])


def _head_dense(x, aggp, degp, Ws, Wn, b, f1, f1b, f2, f2b, f3, f3b):
    wspec = pl.BlockSpec((D, D), lambda i: (0, 0))
    bspec = pl.BlockSpec((1, D), lambda i: (0, 0))
    return pl.pallas_call(
        _head_body,
        grid=(_NB,),
        in_specs=[
            pl.BlockSpec((_RB, D), lambda i: (i, 0)),
            pl.BlockSpec((NC, _RB, D), lambda i: (0, i, 0)),
            pl.BlockSpec((NC, _RB, DW), lambda i: (0, i, 0)),
            wspec, wspec, bspec,
            wspec, bspec, wspec, bspec, wspec, bspec,
        ],
        out_specs=pl.BlockSpec((1, D), lambda i: (0, 0)),
        out_shape=jax.ShapeDtypeStruct((1, D), jnp.float32),
        scratch_shapes=[pltpu.VMEM((1, D), jnp.float32)],
    )(x, aggp, degp, Ws, Wn, b, f1, f1b, f2, f2b, f3, f3b)


def _pad_w(w):
    out = jnp.zeros((D, D), jnp.float32)
    return lax.dynamic_update_slice(out, w, (0, 0))


def _pad_b(b):
    out = jnp.zeros((1, D), jnp.float32)
    return lax.dynamic_update_slice(out, b[None, :], (0, 0))


def kernel(x, edge_index, W1_self, W1_neigh, b1, W2_self, W2_neigh, b2,
           fc1_w, fc1_b, fc2_w, fc2_b, fc3_w, fc3_b):
    npad = EPAD - N_EDGES
    pad_src = (np.arange(npad, dtype=np.int32) * 37) % N_NODES
    pad_dst = N_NODES + (np.arange(npad, dtype=np.int32) % (NPAD - N_NODES))
    src = jnp.concatenate([edge_index[0], jnp.asarray(pad_src)])
    dst = jnp.concatenate([edge_index[1], jnp.asarray(pad_dst)])

    xa = jnp.concatenate(
        [x, jnp.ones((N_NODES, 1), jnp.float32),
         jnp.zeros((N_NODES, DW - D - 1), jnp.float32)], axis=1)

    aggp1 = _seg_sum_aug(src, dst, xa)
    h1 = _sage_dense(x, aggp1, W1_self, W1_neigh, b1[None, :])
    aggp2 = _seg_sum(src, dst, h1)
    out = _head_dense(
        h1, aggp2, aggp1, W2_self, W2_neigh, b2[None, :],
        _pad_w(fc1_w), _pad_b(fc1_b),
        _pad_w(fc2_w), _pad_b(fc2_b),
        _pad_w(fc3_w), _pad_b(fc3_b),
    )
    return out[:, :1]


# speedup vs baseline: 6.5222x; 1.0301x over previous
import functools

import jax
import jax.numpy as jnp
import numpy as np
from jax import lax
from jax.experimental import pallas as pl
from jax.experimental.pallas import tpu as pltpu
from jax.experimental.pallas import tpu_sc as plsc

N_NODES = 10000
N_EDGES = 320000
D = 128
DW = 144

NC = 2
NS = 16
NW = NC * NS

K = 96
EPW = 10176
EPAD = EPW * NW
NCHUNK = EPW // K
NPAD = 10240
RPT = NPAD // NS

_mesh = plsc.VectorSubcoreMesh(core_axis_name="c", subcore_axis_name="s")


def _make_seg_sum(width):
    @functools.partial(
        pl.kernel,
        mesh=_mesh,
        out_type=jax.ShapeDtypeStruct((NC, NPAD, width), jnp.float32),
        scratch_types=[
            pltpu.VMEM_SHARED((NPAD, width), jnp.float32),
            pltpu.VMEM((2, K), jnp.int32),
            pltpu.VMEM((2, K), jnp.int32),
            pltpu.VMEM((2, K, width), jnp.float32),
            pltpu.VMEM((16, width), jnp.float32),
            pltpu.SemaphoreType.DMA,
            pltpu.SemaphoreType.DMA,
        ],
        compiler_params=pltpu.CompilerParams(use_tc_tiling_on_sc=False),
    )
    def seg_sum(src_hbm, dst_hbm, tab_hbm, agg_out,
                acc, idx_s, idx_d, rows, zblk, gsem, ssem):
        c = lax.axis_index("c")
        s = lax.axis_index("s")
        wid = c * NS + s

        zero16 = jnp.zeros((16,), jnp.float32)

        def init_zrow(r, _):
            for j in range(width // 16):
                zblk[r, pl.ds(j * 16, 16)] = zero16
            return 0

        lax.fori_loop(0, 16, init_zrow, 0)

        base_r = s * RPT

        def zero_chunk(i, _):
            pltpu.sync_copy(zblk, acc.at[pl.ds(base_r + i * 16, 16)])
            return 0

        lax.fori_loop(0, RPT // 16, zero_chunk, 0)
        plsc.subcore_barrier()

        base_e = wid * EPW

        pltpu.sync_copy(src_hbm.at[pl.ds(base_e, K)], idx_s.at[0])
        pltpu.sync_copy(dst_hbm.at[pl.ds(base_e, K)], idx_d.at[0])
        pltpu.async_copy(tab_hbm.at[idx_s.at[0]], rows.at[0], gsem)

        def chunk(j, _):
            b = lax.rem(j, 2)
            nb = 1 - b
            pltpu.make_async_copy(
                tab_hbm.at[idx_s.at[b]], rows.at[b], gsem).wait()

            @pl.when(j > 0)
            def _():
                pltpu.make_async_copy(
                    rows.at[nb], acc.at[idx_d.at[nb]], ssem).wait()

            @pl.when(j + 1 < NCHUNK)
            def _():
                off = base_e + (j + 1) * K
                pltpu.sync_copy(src_hbm.at[pl.ds(off, K)], idx_s.at[nb])
                pltpu.sync_copy(dst_hbm.at[pl.ds(off, K)], idx_d.at[nb])
                pltpu.async_copy(tab_hbm.at[idx_s.at[nb]], rows.at[nb], gsem)

            pltpu.async_copy(rows.at[b], acc.at[idx_d.at[b]], ssem, add=True)
            return 0

        lax.fori_loop(0, NCHUNK, chunk, 0)
        lastb = (NCHUNK - 1) % 2
        pltpu.make_async_copy(
            rows.at[lastb], acc.at[idx_d.at[lastb]], ssem).wait()
        plsc.subcore_barrier()

        pltpu.sync_copy(acc.at[pl.ds(base_r, RPT)],
                        agg_out.at[c, pl.ds(base_r, RPT)])

    return seg_sum


_seg_sum_aug = _make_seg_sum(DW)
_seg_sum = _make_seg_sum(D)


_RB = 1000
_NB = N_NODES // _RB


def _sage_body(x_ref, a_ref, ws_ref, wn_ref, b_ref, o_ref):
    deg = jnp.maximum(a_ref[0, :, D:D + 1] + a_ref[1, :, D:D + 1], 1.0)
    mean = (a_ref[0, :, :D] + a_ref[1, :, :D]) / deg
    h = (jnp.dot(x_ref[...], ws_ref[...], preferred_element_type=jnp.float32, precision=lax.Precision.HIGHEST)
         + jnp.dot(mean, wn_ref[...], preferred_element_type=jnp.float32, precision=lax.Precision.HIGHEST)
         + b_ref[...])
    o_ref[...] = jnp.maximum(h, 0.0)


def _sage_dense(x, aggp, Ws, Wn, b):
    return pl.pallas_call(
        _sage_body,
        grid=(_NB,),
        in_specs=[
            pl.BlockSpec((_RB, D), lambda i: (i, 0)),
            pl.BlockSpec((NC, _RB, DW), lambda i: (0, i, 0)),
            pl.BlockSpec((D, D), lambda i: (0, 0)),
            pl.BlockSpec((D, D), lambda i: (0, 0)),
            pl.BlockSpec((1, D), lambda i: (0, 0)),
        ],
        out_specs=pl.BlockSpec((_RB, D), lambda i: (i, 0)),
        out_shape=jax.ShapeDtypeStruct((N_NODES, D), jnp.float32),
    )(x, aggp, Ws, Wn, b)


def _head_body(x_ref, a_ref, d_ref, ws_ref, wn_ref, b_ref,
               f1_ref, f1b_ref, f2_ref, f2b_ref, f3_ref, f3b_ref,
               o_ref, acc_ref):
    i = pl.program_id(0)
    deg = jnp.maximum(d_ref[0, :, D:D + 1] + d_ref[1, :, D:D + 1], 1.0)
    mean = (a_ref[0] + a_ref[1]) / deg
    h = (jnp.dot(x_ref[...], ws_ref[...], preferred_element_type=jnp.float32, precision=lax.Precision.HIGHEST)
         + jnp.dot(mean, wn_ref[...], preferred_element_type=jnp.float32, precision=lax.Precision.HIGHEST)
         + b_ref[...])
    h = jnp.maximum(h, 0.0)

    @pl.when(i == 0)
    def _():
        acc_ref[...] = jnp.zeros_like(acc_ref)

    acc_ref[...] += jnp.sum(h, axis=0, keepdims=True)

    @pl.when(i == _NB - 1)
    def _():
        pooled = acc_ref[...] * (1.0 / N_NODES)
        z = jnp.clip(
            jnp.dot(pooled, f1_ref[...], preferred_element_type=jnp.float32, precision=lax.Precision.HIGHEST)
            + f1b_ref[...], 0.0, 6.0)
        z = jnp.clip(
            jnp.dot(z, f2_ref[...], preferred_element_type=jnp.float32, precision=lax.Precision.HIGHEST)
            + f2b_ref[...], 0.0, 6.0)
        o_ref[...] = (jnp.dot(z, f3_ref[...], preferred_element_type=jnp.float32, precision=lax.Precision.HIGHEST)
                      + f3b_ref[...])


def _head_dense(x, aggp, degp, Ws, Wn, b, f1, f1b, f2, f2b, f3, f3b):
    wspec = pl.BlockSpec((D, D), lambda i: (0, 0))
    bspec = pl.BlockSpec((1, D), lambda i: (0, 0))
    return pl.pallas_call(
        _head_body,
        grid=(_NB,),
        in_specs=[
            pl.BlockSpec((_RB, D), lambda i: (i, 0)),
            pl.BlockSpec((NC, _RB, D), lambda i: (0, i, 0)),
            pl.BlockSpec((NC, _RB, DW), lambda i: (0, i, 0)),
            wspec, wspec, bspec,
            wspec, bspec, wspec, bspec, wspec, bspec,
        ],
        out_specs=pl.BlockSpec((1, D), lambda i: (0, 0)),
        out_shape=jax.ShapeDtypeStruct((1, D), jnp.float32),
        scratch_shapes=[pltpu.VMEM((1, D), jnp.float32)],
    )(x, aggp, degp, Ws, Wn, b, f1, f1b, f2, f2b, f3, f3b)


def _pad_w(w):
    out = jnp.zeros((D, D), jnp.float32)
    return lax.dynamic_update_slice(out, w, (0, 0))


def _pad_b(b):
    out = jnp.zeros((1, D), jnp.float32)
    return lax.dynamic_update_slice(out, b[None, :], (0, 0))


def kernel(x, edge_index, W1_self, W1_neigh, b1, W2_self, W2_neigh, b2,
           fc1_w, fc1_b, fc2_w, fc2_b, fc3_w, fc3_b):
    npad = EPAD - N_EDGES
    pad_src = (np.arange(npad, dtype=np.int32) * 37) % N_NODES
    pad_dst = N_NODES + (np.arange(npad, dtype=np.int32) % (NPAD - N_NODES))
    src = jnp.concatenate([edge_index[0], jnp.asarray(pad_src)])
    dst = jnp.concatenate([edge_index[1], jnp.asarray(pad_dst)])

    xa = jnp.concatenate(
        [x, jnp.ones((N_NODES, 1), jnp.float32),
         jnp.zeros((N_NODES, DW - D - 1), jnp.float32)], axis=1)

    aggp1 = _seg_sum_aug(src, dst, xa)
    h1 = _sage_dense(x, aggp1, W1_self, W1_neigh, b1[None, :])
    aggp2 = _seg_sum(src, dst, h1)
    out = _head_dense(
        h1, aggp2, aggp1, W2_self, W2_neigh, b2[None, :],
        _pad_w(fc1_w), _pad_b(fc1_b),
        _pad_w(fc2_w), _pad_b(fc2_b),
        _pad_w(fc3_w), _pad_b(fc3_b),
    )
    return out[:, :1]


# speedup vs baseline: 9.6333x; 1.4770x over previous
import functools

import jax
import jax.numpy as jnp
import numpy as np
from jax import lax
from jax.experimental import pallas as pl
from jax.experimental.pallas import tpu as pltpu
from jax.experimental.pallas import tpu_sc as plsc

N_NODES = 10000
N_EDGES = 320000
D = 128
DW = 144

NC = 2
NS = 16
NW = NC * NS

K = 112
IB = 4
NCHUNK = 92
NBLK = NCHUNK // IB
EPW = NCHUNK * K
EPAD = EPW * NW
NPAD = 10240
RPT = NPAD // NS

_mesh = plsc.VectorSubcoreMesh(core_axis_name="c", subcore_axis_name="s")


def _make_seg_sum(width):
    @functools.partial(
        pl.kernel,
        mesh=_mesh,
        out_type=jax.ShapeDtypeStruct((NC, NPAD, width), jnp.float32),
        scratch_types=[
            pltpu.VMEM_SHARED((NPAD, width), jnp.float32),
            pltpu.VMEM((2, IB, K), jnp.int32),
            pltpu.VMEM((2, IB, K), jnp.int32),
            pltpu.VMEM((2, K, width), jnp.float32),
            pltpu.VMEM((16, width), jnp.float32),
            pltpu.SemaphoreType.DMA,
            pltpu.SemaphoreType.DMA,
            pltpu.SemaphoreType.DMA,
        ],
        compiler_params=pltpu.CompilerParams(use_tc_tiling_on_sc=False),
    )
    def seg_sum(src_hbm, dst_hbm, tab_hbm, agg_out,
                acc, sidx, didx, rows, zblk, gsem, ssem, isem):
        c = lax.axis_index("c")
        s = lax.axis_index("s")
        wid = c * NS + s

        zero16 = jnp.zeros((16,), jnp.float32)

        def init_zrow(r, _):
            for j in range(width // 16):
                zblk[r, pl.ds(j * 16, 16)] = zero16
            return 0

        lax.fori_loop(0, 16, init_zrow, 0)

        base_r = s * RPT

        def zero_chunk(i, _):
            pltpu.sync_copy(zblk, acc.at[pl.ds(base_r + i * 16, 16)])
            return 0

        lax.fori_loop(0, RPT // 16, zero_chunk, 0)
        plsc.subcore_barrier()

        base_b = wid * NCHUNK

        def idx_start(blk, slot):
            pltpu.async_copy(src_hbm.at[pl.ds(base_b + blk * IB, IB)],
                             sidx.at[slot], isem)
            pltpu.async_copy(dst_hbm.at[pl.ds(base_b + blk * IB, IB)],
                             didx.at[slot], isem)

        def idx_wait(blk, slot):
            pltpu.make_async_copy(src_hbm.at[pl.ds(base_b + blk * IB, IB)],
                                  sidx.at[slot], isem).wait()
            pltpu.make_async_copy(dst_hbm.at[pl.ds(base_b + blk * IB, IB)],
                                  didx.at[slot], isem).wait()

        idx_start(0, 0)
        idx_wait(0, 0)
        pltpu.async_copy(tab_hbm.at[sidx.at[0, 0]], rows.at[0], gsem)

        def blk(g, _):
            slot = lax.rem(g, 2)
            nslot = 1 - slot
            for i in range(IB):
                b = i % 2
                pltpu.make_async_copy(
                    tab_hbm.at[sidx.at[slot, i]], rows.at[b], gsem).wait()
                if i > 0:
                    pltpu.make_async_copy(
                        rows.at[1 - b], acc.at[didx.at[slot, i - 1]],
                        ssem).wait()
                else:
                    @pl.when(g > 0)
                    def _():
                        pltpu.make_async_copy(
                            rows.at[1 - b], acc.at[didx.at[nslot, IB - 1]],
                            ssem).wait()

                    @pl.when(g + 1 < NBLK)
                    def _():
                        idx_start(g + 1, nslot)

                if i == IB - 2:
                    @pl.when(g + 1 < NBLK)
                    def _():
                        idx_wait(g + 1, nslot)

                if i < IB - 1:
                    pltpu.async_copy(tab_hbm.at[sidx.at[slot, i + 1]],
                                     rows.at[1 - b], gsem)
                else:
                    @pl.when(g + 1 < NBLK)
                    def _():
                        pltpu.async_copy(tab_hbm.at[sidx.at[nslot, 0]],
                                         rows.at[1 - b], gsem)

                pltpu.async_copy(rows.at[b], acc.at[didx.at[slot, i]],
                                 ssem, add=True)
            return 0

        lax.fori_loop(0, NBLK, blk, 0)
        pltpu.make_async_copy(
            rows.at[(IB - 1) % 2],
            acc.at[didx.at[(NBLK - 1) % 2, IB - 1]], ssem).wait()
        plsc.subcore_barrier()

        pltpu.sync_copy(acc.at[pl.ds(base_r, RPT)],
                        agg_out.at[c, pl.ds(base_r, RPT)])

    return seg_sum


_seg_sum_aug = _make_seg_sum(DW)
_seg_sum = _make_seg_sum(D)


_RB = 1000
_NB = N_NODES // _RB


def _sage_body(x_ref, a_ref, ws_ref, wn_ref, b_ref, o_ref):
    deg = jnp.maximum(a_ref[0, :, D:D + 1] + a_ref[1, :, D:D + 1], 1.0)
    mean = (a_ref[0, :, :D] + a_ref[1, :, :D]) / deg
    h = (jnp.dot(x_ref[...], ws_ref[...], preferred_element_type=jnp.float32, precision=lax.Precision.HIGHEST)
         + jnp.dot(mean, wn_ref[...], preferred_element_type=jnp.float32, precision=lax.Precision.HIGHEST)
         + b_ref[...])
    o_ref[...] = jnp.maximum(h, 0.0)


def _sage_dense(x, aggp, Ws, Wn, b):
    return pl.pallas_call(
        _sage_body,
        grid=(_NB,),
        in_specs=[
            pl.BlockSpec((_RB, D), lambda i: (i, 0)),
            pl.BlockSpec((NC, _RB, DW), lambda i: (0, i, 0)),
            pl.BlockSpec((D, D), lambda i: (0, 0)),
            pl.BlockSpec((D, D), lambda i: (0, 0)),
            pl.BlockSpec((1, D), lambda i: (0, 0)),
        ],
        out_specs=pl.BlockSpec((_RB, D), lambda i: (i, 0)),
        out_shape=jax.ShapeDtypeStruct((N_NODES, D), jnp.float32),
    )(x, aggp, Ws, Wn, b)


def _head_body(x_ref, a_ref, d_ref, ws_ref, wn_ref, b_ref,
               f1_ref, f1b_ref, f2_ref, f2b_ref, f3_ref, f3b_ref,
               o_ref, acc_ref):
    i = pl.program_id(0)
    deg = jnp.maximum(d_ref[0, :, D:D + 1] + d_ref[1, :, D:D + 1], 1.0)
    mean = (a_ref[0] + a_ref[1]) / deg
    h = (jnp.dot(x_ref[...], ws_ref[...], preferred_element_type=jnp.float32, precision=lax.Precision.HIGHEST)
         + jnp.dot(mean, wn_ref[...], preferred_element_type=jnp.float32, precision=lax.Precision.HIGHEST)
         + b_ref[...])
    h = jnp.maximum(h, 0.0)

    @pl.when(i == 0)
    def _():
        acc_ref[...] = jnp.zeros_like(acc_ref)

    acc_ref[...] += jnp.sum(h, axis=0, keepdims=True)

    @pl.when(i == _NB - 1)
    def _():
        pooled = acc_ref[...] * (1.0 / N_NODES)
        z = jnp.clip(
            jnp.dot(pooled, f1_ref[...], preferred_element_type=jnp.float32, precision=lax.Precision.HIGHEST)
            + f1b_ref[...], 0.0, 6.0)
        z = jnp.clip(
            jnp.dot(z, f2_ref[...], preferred_element_type=jnp.float32, precision=lax.Precision.HIGHEST)
            + f2b_ref[...], 0.0, 6.0)
        o_ref[...] = (jnp.dot(z, f3_ref[...], preferred_element_type=jnp.float32, precision=lax.Precision.HIGHEST)
                      + f3b_ref[...])


def _head_dense(x, aggp, degp, Ws, Wn, b, f1, f1b, f2, f2b, f3, f3b):
    wspec = pl.BlockSpec((D, D), lambda i: (0, 0))
    bspec = pl.BlockSpec((1, D), lambda i: (0, 0))
    return pl.pallas_call(
        _head_body,
        grid=(_NB,),
        in_specs=[
            pl.BlockSpec((_RB, D), lambda i: (i, 0)),
            pl.BlockSpec((NC, _RB, D), lambda i: (0, i, 0)),
            pl.BlockSpec((NC, _RB, DW), lambda i: (0, i, 0)),
            wspec, wspec, bspec,
            wspec, bspec, wspec, bspec, wspec, bspec,
        ],
        out_specs=pl.BlockSpec((1, D), lambda i: (0, 0)),
        out_shape=jax.ShapeDtypeStruct((1, D), jnp.float32),
        scratch_shapes=[pltpu.VMEM((1, D), jnp.float32)],
    )(x, aggp, degp, Ws, Wn, b, f1, f1b, f2, f2b, f3, f3b)


def _pad_w(w):
    out = jnp.zeros((D, D), jnp.float32)
    return lax.dynamic_update_slice(out, w, (0, 0))


def _pad_b(b):
    out = jnp.zeros((1, D), jnp.float32)
    return lax.dynamic_update_slice(out, b[None, :], (0, 0))


def kernel(x, edge_index, W1_self, W1_neigh, b1, W2_self, W2_neigh, b2,
           fc1_w, fc1_b, fc2_w, fc2_b, fc3_w, fc3_b):
    npad = EPAD - N_EDGES
    pad_src = (np.arange(npad, dtype=np.int32) * 37) % N_NODES
    pad_dst = N_NODES + (np.arange(npad, dtype=np.int32) % (NPAD - N_NODES))
    src = jnp.concatenate([edge_index[0],
                           jnp.asarray(pad_src)]).reshape(NW * NCHUNK, K)
    dst = jnp.concatenate([edge_index[1],
                           jnp.asarray(pad_dst)]).reshape(NW * NCHUNK, K)

    xa = jnp.concatenate(
        [x, jnp.ones((N_NODES, 1), jnp.float32),
         jnp.zeros((N_NODES, DW - D - 1), jnp.float32)], axis=1)

    aggp1 = _seg_sum_aug(src, dst, xa)
    h1 = _sage_dense(x, aggp1, W1_self, W1_neigh, b1[None, :])
    aggp2 = _seg_sum(src, dst, h1)
    out = _head_dense(
        h1, aggp2, aggp1, W2_self, W2_neigh, b2[None, :],
        _pad_w(fc1_w), _pad_b(fc1_b),
        _pad_w(fc2_w), _pad_b(fc2_b),
        _pad_w(fc3_w), _pad_b(fc3_b),
    )
    return out[:, :1]
